# baseline (speedup 1.0000x reference)
import functools

import jax
import jax.numpy as jnp
from jax import lax
from jax.experimental import pallas as pl
from jax.experimental.pallas import tpu as pltpu
from jax.experimental.pallas import tpu_sc as plsc

DIM = 64
NUM_NEG = 5
NLOG = 1 + NUM_NEG
CHUNK = 128
LANES = 16


@functools.lru_cache(maxsize=None)
def _build_sc_kernel(B: int, NW: int):
    b_per_w = B // NW
    n_chunks = b_per_w // CHUNK
    mesh = plsc.VectorSubcoreMesh(core_axis_name="c", subcore_axis_name="s")

    @functools.partial(
        pl.kernel,
        mesh=mesh,
        compiler_params=pltpu.CompilerParams(
            use_tc_tiling_on_sc=True, needs_layout_passes=False
        ),
        out_type=jax.ShapeDtypeStruct((B * NLOG,), jnp.float32),
        scratch_types=[
            pltpu.VMEM((CHUNK,), jnp.int32),
            pltpu.VMEM((CHUNK,), jnp.int32),
            pltpu.VMEM((CHUNK * NUM_NEG,), jnp.int32),
            pltpu.VMEM((CHUNK,), jnp.int32),
            pltpu.VMEM((CHUNK,), jnp.int32),
            pltpu.VMEM((CHUNK * NUM_NEG,), jnp.int32),
            pltpu.VMEM((CHUNK, 2 * DIM), jnp.float32),
            pltpu.VMEM((CHUNK, 2 * DIM), jnp.float32),
            pltpu.VMEM((CHUNK * NUM_NEG, 2 * DIM), jnp.float32),
            pltpu.VMEM((CHUNK * NLOG,), jnp.float32),
            pltpu.SemaphoreType.DMA,
        ],
    )
    def sc_kernel(
        cen_hbm, pos_hbm, neg_hbm, table_hbm, out_hbm,
        idc, idp, idn, rowc, rowp, rown, rows_c, rows_p, rows_n, out_v, sem,
    ):
        wid = lax.axis_index("s") * 2 + lax.axis_index("c")
        lane = lax.iota(jnp.int32, LANES)

        for c in range(n_chunks):
            base = wid * b_per_w + c * CHUNK
            pltpu.sync_copy(cen_hbm.at[pl.ds(base, CHUNK)], idc)
            pltpu.sync_copy(pos_hbm.at[pl.ds(base, CHUNK)], idp)
            pltpu.sync_copy(
                neg_hbm.at[pl.ds(base * NUM_NEG, CHUNK * NUM_NEG)], idn
            )

            def halve(i, _):
                s = pl.ds(i * LANES, LANES)
                rowc[s] = idc[s] >> 1
                rowp[s] = idp[s] >> 1
                return 0

            lax.fori_loop(0, CHUNK // LANES, halve, 0)

            def halve_n(i, _):
                s = pl.ds(i * LANES, LANES)
                rown[s] = idn[s] >> 1
                return 0

            lax.fori_loop(0, CHUNK * NUM_NEG // LANES, halve_n, 0)

            cps = [
                pltpu.async_copy(table_hbm.at[rowc], rows_c, sem),
                pltpu.async_copy(table_hbm.at[rowp], rows_p, sem),
            ] + [
                pltpu.async_copy(
                    table_hbm.at[rown.at[pl.ds(g * CHUNK, CHUNK)]],
                    rows_n.at[pl.ds(g * CHUNK, CHUNK)],
                    sem,
                )
                for g in range(NUM_NEG)
            ]
            for cp in cps:
                cp.wait()

            def group(g, _):
                bvec = g * LANES + lane
                s = pl.ds(g * LANES, LANES)
                col_c = (idc[s] & 1) * DIM
                col_p = (idp[s] & 1) * DIM
                nvecs = [bvec * NUM_NEG + j for j in range(NUM_NEG)]
                col_n = [
                    (plsc.load_gather(idn, [nvecs[j]]) & 1) * DIM
                    for j in range(NUM_NEG)
                ]
                acc = [jnp.zeros((LANES,), jnp.float32) for _ in range(NLOG)]
                for d in range(DIM):
                    cen = plsc.load_gather(rows_c, [bvec, col_c + d])
                    acc[0] = acc[0] + cen * plsc.load_gather(
                        rows_p, [bvec, col_p + d]
                    )
                    for j in range(NUM_NEG):
                        acc[1 + j] = acc[1 + j] + cen * plsc.load_gather(
                            rows_n, [nvecs[j], col_n[j] + d]
                        )
                for j in range(NLOG):
                    prob = 1.0 / (1.0 + jnp.exp(-acc[j]))
                    plsc.store_scatter(out_v, [bvec * NLOG + j], prob)
                return 0

            lax.fori_loop(0, CHUNK // LANES, group, 0)

            pltpu.sync_copy(out_v, out_hbm.at[pl.ds(base * NLOG, CHUNK * NLOG)])

    return sc_kernel


def kernel(x_center, x_positive, x_negative, table):
    B = x_center.shape[0]
    NW = 32
    neg_flat = x_negative.reshape(B * NUM_NEG)
    table2 = table.reshape(-1, 2 * DIM)
    flat = _build_sc_kernel(B, NW)(x_center, x_positive, neg_flat, table2)
    return flat.reshape(B, NLOG)
